# Initial kernel scaffold; baseline (speedup 1.0000x reference)
import jax
import jax.numpy as jnp
from jax.experimental import pallas as pl


def kernel(compressed_matrix):
    raise NotImplementedError("write your pallas kernel here")


# speedup vs baseline: 126.0418x; 126.0418x over previous
import jax
import jax.numpy as jnp
from jax import lax
from jax.experimental import pallas as pl
from jax.experimental.pallas import tpu as pltpu
from jax.experimental.pallas import tpu_sc as plsc

N = 4096
L = N * (N - 1) // 2
NW = 32
ROWS_PER_W = N // NW
FRONT = 16
S = N + 16
LANES = 16
NCHUNK = N // LANES


def _body(in_hbm, out_hbm, w_ref, r_ref):
    wid = lax.axis_index("s") * 2 + lax.axis_index("c")
    zeros16 = jnp.zeros((LANES,), jnp.float32)
    iota16 = lax.iota(jnp.int32, LANES)

    def zinit(c, carry):
        r_ref[pl.ds(c * LANES, LANES)] = zeros16
        return carry

    lax.fori_loop(0, (FRONT + N) // LANES, zinit, 0)

    def row(t, carry):
        i = wid + NW * (ROWS_PER_W - 1 - t)
        off = i * N - (i * (i + 1)) // 2
        src = jnp.minimum(off, L - S)
        src = (src // 8) * 8
        d = off - src
        pltpu.sync_copy(in_hbm.at[pl.ds(src, S)], w_ref.at[pl.ds(0, S)])

        cb = (i + 1) // LANES
        shift = d - (i + 1)

        pos0 = cb * LANES + iota16
        idx0 = jnp.maximum(pos0 + shift, 0)
        v0 = plsc.load_gather(w_ref, [idx0])
        v0 = jnp.where(pos0 > i, v0, 0.0)
        r_ref[pl.ds(FRONT + cb * LANES, LANES)] = v0

        def chunk(c, carry2):
            idx = c * LANES + iota16 + shift
            r_ref[pl.ds(FRONT + c * LANES, LANES)] = plsc.load_gather(
                w_ref, [idx])
            return carry2

        lax.fori_loop(cb + 1, NCHUNK, chunk, 0)

        pltpu.sync_copy(r_ref.at[pl.ds(FRONT, N)], out_hbm.at[pl.ds(i * N, N)])
        return carry

    lax.fori_loop(0, ROWS_PER_W, row, 0)


def kernel(compressed_matrix):
    run = pl.kernel(
        _body,
        out_type=jax.ShapeDtypeStruct((N * N,), jnp.float32),
        mesh=plsc.VectorSubcoreMesh(core_axis_name="c", subcore_axis_name="s"),
        compiler_params=pltpu.CompilerParams(needs_layout_passes=False),
        scratch_types=[
            pltpu.VMEM((S,), jnp.float32),
            pltpu.VMEM((FRONT + N,), jnp.float32),
        ],
    )
    return run(compressed_matrix).reshape(N, N)
